# baseline (speedup 1.0000x reference)
import functools

import jax
import jax.numpy as jnp
from jax import lax
from jax.experimental import pallas as pl
from jax.experimental.pallas import tpu as pltpu
from jax.experimental.pallas import tpu_sc as plsc

B = 16384
NC = 2
NS = 16
NW = NC * NS
B_W = B // NW
CH = 128
NCHUNK = B_W // CH
L = 16
NV = B_W // L


def _mesh():
    return plsc.VectorSubcoreMesh(core_axis_name="c", subcore_axis_name="s")


def _sqrt16(z):
    f32 = jnp.float32
    w = jnp.maximum(z, f32(2.0 ** -126))
    s = jnp.full_like(w, f32(1.0))
    for p in (64, 32, 16, 8, 4, 2):
        big = w >= f32(2.0 ** p)
        w = jnp.where(big, w * f32(2.0 ** -p), w)
        s = jnp.where(big, s * f32(2.0 ** (p // 2)), s)
        small = w < f32(2.0 ** -p)
        w = jnp.where(small, w * f32(2.0 ** p), w)
        s = jnp.where(small, s * f32(2.0 ** -(p // 2)), s)
    big = w >= f32(2.0)
    w = jnp.where(big, w * f32(0.5), w)
    s = jnp.where(big, s * f32(1.4142135623730951), s)
    small = w < f32(0.5)
    w = jnp.where(small, w + w, w)
    s = jnp.where(small, s * f32(0.7071067811865476), s)
    y = f32(2.00693) + w * (f32(-1.36395) + w * f32(0.35702))
    for _ in range(3):
        y = y * (f32(1.5) - f32(0.5) * w * y * y)
    return w * y * s


@functools.partial(
    pl.kernel,
    mesh=_mesh(),
    out_type=jax.ShapeDtypeStruct((B,), jnp.float32),
    scratch_types=[
        pltpu.VMEM((NCHUNK, CH), jnp.int32),
        pltpu.VMEM((NCHUNK, CH), jnp.int32),
        pltpu.VMEM((NCHUNK, CH), jnp.int32),
        pltpu.VMEM((NCHUNK, CH), jnp.int32),
        pltpu.VMEM((NCHUNK, CH), jnp.int32),
        pltpu.VMEM((NCHUNK, CH), jnp.int32),
        pltpu.VMEM((NCHUNK, CH), jnp.int32),
        pltpu.VMEM((NCHUNK, CH), jnp.int32),
        pltpu.VMEM((B_W,), jnp.float32),
        pltpu.VMEM((B_W,), jnp.float32),
        pltpu.VMEM((B_W,), jnp.float32),
        pltpu.VMEM((B_W,), jnp.float32),
        pltpu.VMEM((B_W,), jnp.float32),
        pltpu.VMEM((B_W,), jnp.float32),
        pltpu.VMEM((B_W,), jnp.float32),
        pltpu.VMEM((B_W,), jnp.float32),
        pltpu.SemaphoreType.DMA,
    ],
)
def _ideal_point_sc(leg_hbm, vote_hbm, xf_hbm, af_hbm, b_hbm, out_hbm,
                    leg_v, vote_v, lx0, lx1, lx2, vx0, vx1, vx2,
                    x0_v, x1_v, x2_v, a0_v, a1_v, a2_v, bg_v, out_v, sem):
    wid = lax.axis_index("s") * NC + lax.axis_index("c")
    base = wid * B_W

    for j in range(NCHUNK):
        pltpu.sync_copy(leg_hbm.at[pl.ds(base + j * CH, CH)], leg_v.at[j])
        pltpu.sync_copy(vote_hbm.at[pl.ds(base + j * CH, CH)], vote_v.at[j])

    three = jnp.full((L,), 3, jnp.int32)
    one = jnp.ones((L,), jnp.int32)

    def ixstep(i):
        j = i // (CH // L)
        o = (i % (CH // L)) * L
        lg = leg_v[j, pl.ds(o, L)] * three
        vt = vote_v[j, pl.ds(o, L)] * three
        lx0[j, pl.ds(o, L)] = lg
        lx1[j, pl.ds(o, L)] = lg + one
        lx2[j, pl.ds(o, L)] = lg + one + one
        vx0[j, pl.ds(o, L)] = vt
        vx1[j, pl.ds(o, L)] = vt + one
        vx2[j, pl.ds(o, L)] = vt + one + one

    for i in range(NV):
        ixstep(i)

    copies = []
    for j in range(NCHUNK):
        sl = pl.ds(j * CH, CH)
        copies.append(pltpu.async_copy(xf_hbm.at[lx0.at[j]], x0_v.at[sl], sem))
        copies.append(pltpu.async_copy(xf_hbm.at[lx1.at[j]], x1_v.at[sl], sem))
        copies.append(pltpu.async_copy(xf_hbm.at[lx2.at[j]], x2_v.at[sl], sem))
        copies.append(pltpu.async_copy(af_hbm.at[vx0.at[j]], a0_v.at[sl], sem))
        copies.append(pltpu.async_copy(af_hbm.at[vx1.at[j]], a1_v.at[sl], sem))
        copies.append(pltpu.async_copy(af_hbm.at[vx2.at[j]], a2_v.at[sl], sem))
        copies.append(pltpu.async_copy(b_hbm.at[vote_v.at[j]], bg_v.at[sl], sem))
    for c in copies:
        c.wait()

    def step(i):
        sl = pl.ds(i * L, L)
        bb = bg_v[sl]
        d0 = x0_v[sl] - bb
        d1 = x1_v[sl] - bb
        d2 = x2_v[sl] - bb
        sd = d0 * d0 + d1 * d1 + d2 * d2
        a0 = a0_v[sl]
        a1 = a1_v[sl]
        a2 = a2_v[sl]
        sa = a0 * a0 + a1 * a1 + a2 * a2
        z = sd * sa

        t = _sqrt16(z)

        out_v[sl] = jnp.float32(1.0) / (jnp.float32(1.0) + jnp.exp(-t))

    for i in range(NV):
        step(i)

    pltpu.sync_copy(out_v, out_hbm.at[pl.ds(base, B_W)])


def kernel(leg_ids, vote_ids, x, a, b):
    return _ideal_point_sc(
        leg_ids.astype(jnp.int32),
        vote_ids.astype(jnp.int32),
        x.astype(jnp.float32).reshape(-1),
        a.astype(jnp.float32).reshape(-1),
        b.astype(jnp.float32),
    )
